# Initial kernel scaffold; baseline (speedup 1.0000x reference)
import jax
import jax.numpy as jnp
from jax.experimental import pallas as pl


def kernel(inputs, bass_w1, bass_b1, bass_w2, bass_b2, bass_w3, bass_b3, alto_w1, alto_b1, alto_w2, alto_b2, alto_w3, alto_b3, tenor_w1, tenor_b1, tenor_w2, tenor_b2, tenor_w3, tenor_b3):
    raise NotImplementedError("write your pallas kernel here")


# speedup vs baseline: 32.8174x; 32.8174x over previous
import functools
import jax
import jax.numpy as jnp
from jax.experimental import pallas as pl

P = 512
H = 256
IN = 81 * 512 + 33 * 2
CHUNK = 8192
NCH = 5
KMAIN = NCH * CHUNK
TAIL = IN - KMAIN
TAILP = 640

_SELU_ALPHA = 1.6732632423543772848170429916717
_SELU_SCALE = 1.0507009873554804934193349852946


def _selu(x):
    return jnp.where(x > 0, _SELU_SCALE * x,
                     (_SELU_SCALE * _SELU_ALPHA) * (jnp.exp(x) - 1.0))


def _logsm_rows(x):
    m = jnp.max(x, axis=1, keepdims=True)
    e = jnp.exp(x - m)
    s = jnp.sum(e, axis=1, keepdims=True)
    return jnp.log(e / s)


def _lex_first(av, ai, bv, bi):
    return (av > bv) | ((av == bv) & (ai < bi))


def _col_stage(v, ix, k, j):
    n, L = v.shape
    g = n // (2 * j)
    v4 = v.reshape(g, 2, j, L)
    i4 = ix.reshape(g, 2, j, L)
    av, bv = v4[:, 0], v4[:, 1]
    ai, bi = i4[:, 0], i4[:, 1]
    a_first = _lex_first(av, ai, bv, bi)
    gidx = jax.lax.broadcasted_iota(jnp.int32, (g, j, L), 0)
    desc = ((gidx * (2 * j)) // k) % 2 == 0
    keep_a = desc == a_first
    nav = jnp.where(keep_a, av, bv)
    nbv = jnp.where(keep_a, bv, av)
    nai = jnp.where(keep_a, ai, bi)
    nbi = jnp.where(keep_a, bi, ai)
    v = jnp.stack([nav, nbv], axis=1).reshape(n, L)
    ix = jnp.stack([nai, nbi], axis=1).reshape(n, L)
    return v, ix


def _col_sort(v, ix):
    n = v.shape[0]
    k = 2
    while k <= n:
        j = k // 2
        while j >= 1:
            v, ix = _col_stage(v, ix, k, j)
            j //= 2
        k *= 2
    return v, ix


def _col_merge_desc(v, ix):
    n = v.shape[0]
    j = n // 2
    while j >= 1:
        v, ix = _col_stage(v, ix, 2 * n, j)
        j //= 2
    return v, ix


def _flip0(x):
    n = x.shape[0]
    j = n // 2
    while j >= 1:
        s = x.reshape(n // (2 * j), 2, j, *x.shape[1:])
        x = jnp.stack([s[:, 1], s[:, 0]], axis=1).reshape(x.shape)
        j //= 2
    return x


def _global_top512(v, ix):
    v, ix = _col_sort(v, ix)
    L = v.shape[1]
    while L > 1:
        half = L // 2
        av, bv = v[:, :half], v[:, half:]
        ai, bi = ix[:, :half], ix[:, half:]
        bv = _flip0(bv)
        bi = _flip0(bi)
        v = jnp.concatenate([av, bv], axis=0)
        ix = jnp.concatenate([ai, bi], axis=0)
        v, ix = _col_merge_desc(v, ix)
        v, ix = v[:P], ix[:P]
        L = half
    return v, ix


def _lane_roll(x, s):
    return jnp.concatenate([x[:, s:], x[:, :s]], axis=1)


def _lane_sort_desc(v, ix):
    n = v.shape[1]
    lane = jax.lax.broadcasted_iota(jnp.int32, v.shape, 1)
    k = 2
    while k <= n:
        j = k // 2
        while j >= 1:
            i_lo0 = (lane & j) == 0
            pv = jnp.where(i_lo0, _lane_roll(v, j), _lane_roll(v, n - j))
            pi = jnp.where(i_lo0, _lane_roll(ix, j), _lane_roll(ix, n - j))
            gt = _lex_first(v, ix, pv, pi)
            i_lo = (lane & j) == 0
            desc = (lane // k) % 2 == 0
            keep = i_lo == (desc == gt)
            v = jnp.where(keep, v, pv)
            ix = jnp.where(keep, ix, pi)
            j //= 2
        k *= 2
    return v, ix


def _pc1_body(v_ref, bw_ref, aw_ref, tw_ref, ob_ref, oa_ref, ot_ref):
    @pl.when(pl.program_id(0) == 0)
    def _():
        ob_ref[...] = jnp.zeros_like(ob_ref)
        oa_ref[...] = jnp.zeros_like(oa_ref)
        ot_ref[...] = jnp.zeros_like(ot_ref)
    vb = v_ref[...]
    ob_ref[...] += jnp.dot(vb, bw_ref[...], preferred_element_type=jnp.float32)
    oa_ref[...] += jnp.dot(vb, aw_ref[...], preferred_element_type=jnp.float32)
    ot_ref[...] += jnp.dot(vb, tw_ref[...], preferred_element_type=jnp.float32)


def _pc1(v2d, bw1, aw1, tw1):
    out = jax.ShapeDtypeStruct((1, H), jnp.float32)
    return pl.pallas_call(
        _pc1_body,
        grid=(NCH,),
        in_specs=[
            pl.BlockSpec((1, CHUNK), lambda k: (0, k)),
            pl.BlockSpec((CHUNK, H), lambda k: (k, 0)),
            pl.BlockSpec((CHUNK, H), lambda k: (k, 0)),
            pl.BlockSpec((CHUNK, H), lambda k: (k, 0)),
        ],
        out_specs=[
            pl.BlockSpec((1, H), lambda k: (0, 0)),
            pl.BlockSpec((1, H), lambda k: (0, 0)),
            pl.BlockSpec((1, H), lambda k: (0, 0)),
        ],
        out_shape=[out, out, out],
    )(v2d, bw1, aw1, tw1)


def _pc2_body(base_ref, vt_ref, wt_ref, b1_ref, w2_ref, b2_ref, w3_ref, b3_ref,
              r0_ref, r1_ref):
    pre = base_ref[...] + jnp.dot(vt_ref[...], wt_ref[...],
                                  preferred_element_type=jnp.float32) + b1_ref[...]
    h1 = _selu(pre)
    h2 = _selu(jnp.dot(h1, w2_ref[...], preferred_element_type=jnp.float32) + b2_ref[...])
    out = jnp.dot(h2, w3_ref[...], preferred_element_type=jnp.float32) + b3_ref[...]
    logp = _logsm_rows(out)
    idx0 = jax.lax.broadcasted_iota(jnp.int32, logp.shape, 1)
    sv, si = _lane_sort_desc(logp, idx0)
    r0_ref[...] = sv
    r1_ref[...] = si


def _pc2(base_b, v_tail, wb_tail, bb1, bw2, bb2, bw3, bb3):
    return pl.pallas_call(
        _pc2_body,
        out_shape=[jax.ShapeDtypeStruct((1, P), jnp.float32),
                   jax.ShapeDtypeStruct((1, P), jnp.int32)],
    )(base_b, v_tail, wb_tail, bb1, bw2, bb2, bw3, bb3)


def _head_scores(base, vt, wt, b1, w2, b2, w3, b3, oh_cols):
    lane = jax.lax.broadcasted_iota(jnp.int32, (P, P), 1)
    xs = [jnp.broadcast_to(vt, (P, TAILP))]
    for c in oh_cols:
        xs.append(jnp.where(lane == c, 1.0, 0.0).astype(jnp.float32))
    x = jnp.concatenate(xs, axis=1)
    pre = (base + jnp.dot(x, wt, preferred_element_type=jnp.float32)) + b1
    h1 = _selu(pre)
    h2 = _selu(jnp.dot(h1, w2, preferred_element_type=jnp.float32) + b2)
    out = jnp.dot(h2, w3, preferred_element_type=jnp.float32) + b3
    return _logsm_rows(out)


def _gather_row(row_vals, hist_col):
    lane = jax.lax.broadcasted_iota(jnp.int32, (P, P), 1)
    m = lane == hist_col
    return jnp.sum(jnp.where(m, jnp.broadcast_to(row_vals, (P, P)), 0.0),
                   axis=1, keepdims=True)


def _pc3_body(base_ref, vt_ref, wt_ref, b1_ref, w2_ref, b2_ref, w3_ref, b3_ref,
              r0c_ref, r1row_ref,
              o0_ref, o1_ref, o2_ref):
    r1col = r1row_ref[...].astype(jnp.float32)
    r1c = _gather_row(r1col, jax.lax.broadcasted_iota(jnp.int32, (P, 1), 0))
    r0c = _gather_row(r0c_ref[...], jax.lax.broadcasted_iota(jnp.int32, (P, 1), 0))
    logp = _head_scores(base_ref[...], vt_ref[...], wt_ref[...], b1_ref[...],
                        w2_ref[...], b2_ref[...], w3_ref[...], b3_ref[...],
                        [r1c.astype(jnp.int32)])
    flat = logp + r0c
    idx = jax.lax.broadcasted_iota(jnp.int32, (P, P), 0) * P + \
        jax.lax.broadcasted_iota(jnp.int32, (P, P), 1)
    sv, si = _global_top512(flat, idx)
    hist = si // P
    res2 = (si % P).astype(jnp.float32)
    res1n = _gather_row(r1row_ref[...].astype(jnp.float32), hist)
    o0_ref[...] = jnp.broadcast_to(sv, (P, 128))
    o1_ref[...] = jnp.broadcast_to(res1n, (P, 128))
    o2_ref[...] = jnp.broadcast_to(res2, (P, 128))


def _pc4_body(base_ref, vt_ref, wt_ref, b1_ref, w2_ref, b2_ref, w3_ref, b3_ref,
              r0row_ref, r1row_ref, r2row_ref,
              o0_ref, o1_ref, o2_ref, o3_ref):
    rows = jax.lax.broadcasted_iota(jnp.int32, (P, 1), 0)
    r0c = _gather_row(r0row_ref[...], rows)
    r1c = _gather_row(r1row_ref[...], rows)
    r2c = _gather_row(r2row_ref[...], rows)
    logp = _head_scores(base_ref[...], vt_ref[...], wt_ref[...], b1_ref[...],
                        w2_ref[...], b2_ref[...], w3_ref[...], b3_ref[...],
                        [r1c.astype(jnp.int32), r2c.astype(jnp.int32)])
    flat = logp + r0c
    idx = jax.lax.broadcasted_iota(jnp.int32, (P, P), 0) * P + \
        jax.lax.broadcasted_iota(jnp.int32, (P, P), 1)
    sv, si = _global_top512(flat, idx)
    hist = si // P
    res3 = (si % P).astype(jnp.float32)
    res1n = _gather_row(r1row_ref[...], hist)
    res2n = _gather_row(r2row_ref[...], hist)
    o0_ref[...] = jnp.broadcast_to(sv, (P, 128))
    o1_ref[...] = jnp.broadcast_to(res1n, (P, 128))
    o2_ref[...] = jnp.broadcast_to(res2n, (P, 128))
    o3_ref[...] = jnp.broadcast_to(res3, (P, 128))


def _outP(n):
    return [jax.ShapeDtypeStruct((P, 128), jnp.float32) for _ in range(n)]


def _pad_tail(w, blocks):
    parts = [w[KMAIN:IN], jnp.zeros((TAILP - TAIL, H), jnp.float32)]
    for s, n in blocks:
        parts.append(w[s:s + n])
    return jnp.concatenate(parts, axis=0)


def kernel(inputs, bass_w1, bass_b1, bass_w2, bass_b2, bass_w3, bass_b3,
           alto_w1, alto_b1, alto_w2, alto_b2, alto_w3, alto_b3,
           tenor_w1, tenor_b1, tenor_w2, tenor_b2, tenor_w3, tenor_b3):
    v2d = inputs.reshape(1, IN)
    v_tail = jnp.concatenate(
        [inputs[KMAIN:], jnp.zeros((TAILP - TAIL,), jnp.float32)]).reshape(1, TAILP)

    base_b, base_a, base_t = _pc1(v2d, bass_w1, alto_w1[:IN], tenor_w1[:IN])

    wb_tail = jnp.concatenate(
        [bass_w1[KMAIN:IN], jnp.zeros((TAILP - TAIL, H), jnp.float32)], axis=0)
    r0_row, r1_row = _pc2(base_b, v_tail, wb_tail,
                          bass_b1.reshape(1, H), bass_w2, bass_b2.reshape(1, H),
                          bass_w3, bass_b3.reshape(1, P))

    aw_tail = _pad_tail(alto_w1, [(IN, P)])
    a0, a1, a2 = pl.pallas_call(
        _pc3_body, out_shape=_outP(3),
    )(base_a, v_tail, aw_tail, alto_b1.reshape(1, H), alto_w2,
      alto_b2.reshape(1, H), alto_w3, alto_b3.reshape(1, P),
      r0_row, r1_row)

    tw_tail = _pad_tail(tenor_w1, [(IN, P), (IN + P, P)])
    t0, t1, t2, t3 = pl.pallas_call(
        _pc4_body, out_shape=_outP(4),
    )(base_t, v_tail, tw_tail, tenor_b1.reshape(1, H), tenor_w2,
      tenor_b2.reshape(1, H), tenor_w3, tenor_b3.reshape(1, P),
      a0[:, 0].reshape(1, P), a1[:, 0].reshape(1, P), a2[:, 0].reshape(1, P))

    return jnp.stack([t0[:, 0], t1[:, 0], t2[:, 0], t3[:, 0]], axis=1)
